# baseline (speedup 1.0000x reference)
import functools

import jax
import jax.numpy as jnp
from jax import lax
from jax.experimental import pallas as pl
from jax.experimental.pallas import tpu as pltpu
from jax.experimental.pallas import tpu_sc as plsc

S, B, D = 2048, 2, 1024
E, CAP, DFF, FH = 64, 128, 512, 2048
T = S * B
TB = 512
NBLK = T // TB
BUF_ROWS = E * CAP + CAP
DUMP = E * CAP

NC, NS = 2, 16
NW = NC * NS
CHUNK = 64
NCHUNK = T // CHUNK
CPW = NCHUNK // NW


def _stage_a(x_ref, res_ref, w1_ref, b1_ref, w2_ref, b2_ref, g_ref, b_ref,
             wg_ref, xn_ref, sc_ref, ss_ref, sl_ref, cf_ref, la_ref,
             cnt_ref, ce_ref):
    i = pl.program_id(0)

    @pl.when(i == 0)
    def _init():
        cnt_ref[...] = jnp.zeros((1, E), jnp.float32)
        ce_ref[...] = jnp.zeros((1, E), jnp.float32)

    xb = x_ref[...]
    h1 = jnp.maximum(
        jnp.dot(xb, w1_ref[...], preferred_element_type=jnp.float32)
        + b1_ref[...], 0.0)
    resb = res_ref[...]
    res_il = jnp.stack([resb[0], resb[1]], axis=1).reshape(TB, D)
    h = (jnp.dot(h1, w2_ref[...], preferred_element_type=jnp.float32)
         + b2_ref[...] + res_il)
    sc_ref[...] = h

    mu = jnp.mean(h, axis=1, keepdims=True)
    var = jnp.mean((h - mu) ** 2, axis=1, keepdims=True)
    xn = (h - mu) * lax.rsqrt(var + 1e-5) * g_ref[...] + b_ref[...]
    xn_ref[...] = xn

    logits = jnp.dot(xn, wg_ref[...], preferred_element_type=jnp.float32)
    m = jnp.max(logits, axis=1, keepdims=True)
    ex = jnp.exp(logits - m)
    gates = ex / jnp.sum(ex, axis=1, keepdims=True)
    gate = jnp.max(gates, axis=1, keepdims=True)

    iota_e = lax.broadcasted_iota(jnp.int32, (TB, E), 1)
    eq = logits == m
    eidx = jnp.min(jnp.where(eq, iota_e, E), axis=1, keepdims=True)
    onehot = iota_e == eidx
    oh = onehot.astype(jnp.float32)

    ir = lax.broadcasted_iota(jnp.int32, (TB, TB), 0)
    ic = lax.broadcasted_iota(jnp.int32, (TB, TB), 1)
    tri = (ir >= ic).astype(jnp.float32)
    csum = jnp.dot(tri, oh, preferred_element_type=jnp.float32)

    pos_f = csum + cnt_ref[...] - 1.0
    pos = jnp.sum(jnp.where(onehot, pos_f, 0.0), axis=1, keepdims=True)
    keep = pos < float(CAP)
    posc = jnp.minimum(pos, float(CAP - 1)).astype(jnp.int32)
    slot_c = eidx * CAP + posc
    ss_ref[...] = jnp.where(keep, slot_c, DUMP)
    sl_ref[...] = slot_c
    cf_ref[...] = jnp.broadcast_to(jnp.where(keep, gate, 0.0), (TB, LANES))

    oh_sum = jnp.sum(oh, axis=0, keepdims=True)
    cnt_new = cnt_ref[...] + oh_sum
    ce_new = ce_ref[...] + jnp.sum(gates, axis=0, keepdims=True)
    cnt_ref[...] = cnt_new
    ce_ref[...] = ce_new

    @pl.when(i == NBLK - 1)
    def _laux():
        me = cnt_new / float(T)
        ce = ce_new / float(T)
        la_ref[...] = float(E) * jnp.sum(me * ce, axis=1, keepdims=True)


def _run_stage_a(xf, resf, w1, b1, w2, b2, ln_g, ln_b, wg):
    return pl.pallas_call(
        _stage_a,
        grid=(NBLK,),
        in_specs=[
            pl.BlockSpec((TB, D), lambda i: (i, 0)),
            pl.BlockSpec((B, TB // B, D), lambda i: (0, i, 0)),
            pl.BlockSpec((D, FH), lambda i: (0, 0)),
            pl.BlockSpec((1, FH), lambda i: (0, 0)),
            pl.BlockSpec((FH, D), lambda i: (0, 0)),
            pl.BlockSpec((1, D), lambda i: (0, 0)),
            pl.BlockSpec((1, D), lambda i: (0, 0)),
            pl.BlockSpec((1, D), lambda i: (0, 0)),
            pl.BlockSpec((D, E), lambda i: (0, 0)),
        ],
        out_specs=[
            pl.BlockSpec((TB, D), lambda i: (i, 0)),
            pl.BlockSpec((TB, D), lambda i: (i, 0)),
            pl.BlockSpec((TB, 1), lambda i: (i, 0)),
            pl.BlockSpec((TB, 1), lambda i: (i, 0)),
            pl.BlockSpec((TB, 16), lambda i: (i, 0)),
            pl.BlockSpec((1, 1), lambda i: (0, 0)),
        ],
        out_shape=[
            jax.ShapeDtypeStruct((T, D), jnp.float32),
            jax.ShapeDtypeStruct((T, D), jnp.float32),
            jax.ShapeDtypeStruct((T, 1), jnp.int32),
            jax.ShapeDtypeStruct((T, 1), jnp.int32),
            jax.ShapeDtypeStruct((T, 16), jnp.float32),
            jax.ShapeDtypeStruct((1, 1), jnp.float32),
        ],
        scratch_shapes=[
            pltpu.VMEM((1, E), jnp.float32),
            pltpu.VMEM((1, E), jnp.float32),
        ],
    )(xf, resf, w1, b1, w2, b2, ln_g, ln_b, wg)


def _sc_scatter(xn_hbm, idx_hbm, buf_hbm, idx_v, rows_v, sem):
    wid = lax.axis_index("s") * NC + lax.axis_index("c")
    for j in range(CPW):
        c = wid * CPW + j
        pltpu.sync_copy(idx_hbm.at[c], idx_v)
        pltpu.sync_copy(xn_hbm.at[pl.ds(c * CHUNK, CHUNK)], rows_v)
        pltpu.async_copy(rows_v, buf_hbm.at[idx_v], sem).wait()


@functools.lru_cache(maxsize=None)
def _sc_scatter_call():
    return pl.kernel(
        _sc_scatter,
        mesh=plsc.VectorSubcoreMesh(core_axis_name="c", subcore_axis_name="s"),
        out_type=jax.ShapeDtypeStruct((BUF_ROWS, D), jnp.float32),
        scratch_types=[
            pltpu.VMEM((CHUNK,), jnp.int32),
            pltpu.VMEM((CHUNK, D), jnp.float32),
            pltpu.SemaphoreType.DMA,
        ],
    )


GCH = 32
NGC = T // GCH
GPW = NGC // NW
LANES = 16
NGRP = D // LANES


def _sc_gather(oe_hbm, idx_hbm, cf_hbm, sh_hbm, out_hbm,
               idx_v, rows_v, sh_v, cf_v, sem):
    wid = lax.axis_index("s") * NC + lax.axis_index("c")
    for j in range(GPW):
        c = wid * GPW + j
        pltpu.sync_copy(idx_hbm.at[c], idx_v)
        pltpu.sync_copy(cf_hbm.at[pl.ds(c * GCH, GCH)], cf_v)
        pltpu.sync_copy(sh_hbm.at[pl.ds(c * GCH, GCH)], sh_v)
        pltpu.async_copy(oe_hbm.at[idx_v], rows_v, sem).wait()

        def _row(r, carry):
            cf = cf_v[r]
            for g in range(NGRP):
                ds = pl.ds(g * LANES, LANES)
                sh_v[r, ds] = rows_v[r, ds] * cf + sh_v[r, ds]
            return carry

        lax.fori_loop(0, GCH, _row, 0)
        pltpu.sync_copy(sh_v, out_hbm.at[pl.ds(c * GCH, GCH)])


@functools.lru_cache(maxsize=None)
def _sc_gather_call():
    return pl.kernel(
        _sc_gather,
        mesh=plsc.VectorSubcoreMesh(core_axis_name="c", subcore_axis_name="s"),
        out_type=jax.ShapeDtypeStruct((T, D), jnp.float32),
        scratch_types=[
            pltpu.VMEM((GCH,), jnp.int32),
            pltpu.VMEM((GCH, D), jnp.float32),
            pltpu.VMEM((GCH, D), jnp.float32),
            pltpu.VMEM((GCH, LANES), jnp.float32),
            pltpu.SemaphoreType.DMA,
        ],
    )


def _stage_d(buf_ref, w1_ref, b1_ref, w2_ref, b2_ref, oe_ref):
    xb = buf_ref[...]
    w1e = w1_ref[...].reshape(D, DFF)
    w2e = w2_ref[...].reshape(DFF, D)
    h = jnp.maximum(
        jnp.dot(xb, w1e, preferred_element_type=jnp.float32)
        + b1_ref[...].reshape(1, DFF), 0.0)
    oe_ref[...] = (jnp.dot(h, w2e, preferred_element_type=jnp.float32)
                   + b2_ref[...].reshape(1, D))


def _run_stage_d(buf, we1, be1, we2, be2):
    return pl.pallas_call(
        _stage_d,
        grid=(E,),
        in_specs=[
            pl.BlockSpec((CAP, D), lambda e: (e, 0)),
            pl.BlockSpec((1, D, DFF), lambda e: (e, 0, 0)),
            pl.BlockSpec((1, 1, DFF), lambda e: (e, 0, 0)),
            pl.BlockSpec((1, DFF, D), lambda e: (e, 0, 0)),
            pl.BlockSpec((1, 1, D), lambda e: (e, 0, 0)),
        ],
        out_specs=[pl.BlockSpec((CAP, D), lambda e: (e, 0))],
        out_shape=[jax.ShapeDtypeStruct((E * CAP, D), jnp.float32)],
    )(buf, we1, be1, we2, be2)[0]


def kernel(x, residual, w1, b1, w2, b2, ln_g, ln_b, wg, we1, be1, we2, be2):
    xf = x.reshape(T, D)

    xn, shortcut, slot_s, slot_c, coef, la = _run_stage_a(
        xf, residual, w1, b1.reshape(1, FH), w2, b2.reshape(1, D),
        ln_g.reshape(1, D), ln_b.reshape(1, D), wg)

    buf = _sc_scatter_call()(xn, slot_s.reshape(NCHUNK, CHUNK))
    oe = _run_stage_d(buf, we1, be1.reshape(E, 1, DFF),
                      we2, be2.reshape(E, 1, D))
    out = _sc_gather_call()(oe, slot_c.reshape(NGC, GCH), coef, shortcut)

    return out.reshape(S, B, D), la.reshape(())


# speedup vs baseline: 1.0214x; 1.0214x over previous
import functools

import jax
import jax.numpy as jnp
from jax import lax
from jax.experimental import pallas as pl
from jax.experimental.pallas import tpu as pltpu
from jax.experimental.pallas import tpu_sc as plsc

S, B, D = 2048, 2, 1024
E, CAP, DFF, FH = 64, 128, 512, 2048
T = S * B
TB = 512
NBLK = T // TB
BUF_ROWS = E * CAP + CAP
DUMP = E * CAP

NC, NS = 2, 16
NW = NC * NS
CHUNK = 64
NCHUNK = T // CHUNK
CPW = NCHUNK // NW


def _stage_a(x_ref, res_ref, w1_ref, b1_ref, w2_ref, b2_ref, g_ref, b_ref,
             wg_ref, xn_ref, sc_ref, ss_ref, sl_ref, cf_ref, la_ref,
             cnt_ref, ce_ref):
    i = pl.program_id(0)

    @pl.when(i == 0)
    def _init():
        cnt_ref[...] = jnp.zeros((1, E), jnp.float32)
        ce_ref[...] = jnp.zeros((1, E), jnp.float32)

    xb = x_ref[...]
    h1 = jnp.maximum(
        jnp.dot(xb, w1_ref[...], preferred_element_type=jnp.float32)
        + b1_ref[...], 0.0)
    resb = res_ref[...]
    res_il = jnp.stack([resb[0], resb[1]], axis=1).reshape(TB, D)
    h = (jnp.dot(h1, w2_ref[...], preferred_element_type=jnp.float32)
         + b2_ref[...] + res_il)
    sc_ref[...] = h

    mu = jnp.mean(h, axis=1, keepdims=True)
    var = jnp.mean((h - mu) ** 2, axis=1, keepdims=True)
    xn = (h - mu) * lax.rsqrt(var + 1e-5) * g_ref[...] + b_ref[...]
    xn_ref[...] = xn

    logits = jnp.dot(xn, wg_ref[...], preferred_element_type=jnp.float32)
    m = jnp.max(logits, axis=1, keepdims=True)
    ex = jnp.exp(logits - m)
    gates = ex / jnp.sum(ex, axis=1, keepdims=True)
    gate = jnp.max(gates, axis=1, keepdims=True)

    iota_e = lax.broadcasted_iota(jnp.int32, (TB, E), 1)
    eq = logits == m
    eidx = jnp.min(jnp.where(eq, iota_e, E), axis=1, keepdims=True)
    onehot = iota_e == eidx
    oh = onehot.astype(jnp.float32)

    ir = lax.broadcasted_iota(jnp.int32, (TB, TB), 0)
    ic = lax.broadcasted_iota(jnp.int32, (TB, TB), 1)
    tri = (ir >= ic).astype(jnp.float32)
    csum = jnp.dot(tri, oh, preferred_element_type=jnp.float32)

    pos_f = csum + cnt_ref[...] - 1.0
    pos = jnp.sum(jnp.where(onehot, pos_f, 0.0), axis=1, keepdims=True)
    keep = pos < float(CAP)
    posc = jnp.minimum(pos, float(CAP - 1)).astype(jnp.int32)
    slot_c = eidx * CAP + posc
    ss_ref[...] = jnp.where(keep, slot_c, DUMP)
    sl_ref[...] = slot_c
    cf_ref[...] = jnp.where(keep, gate, 0.0)

    oh_sum = jnp.sum(oh, axis=0, keepdims=True)
    cnt_new = cnt_ref[...] + oh_sum
    ce_new = ce_ref[...] + jnp.sum(gates, axis=0, keepdims=True)
    cnt_ref[...] = cnt_new
    ce_ref[...] = ce_new

    @pl.when(i == NBLK - 1)
    def _laux():
        me = cnt_new / float(T)
        ce = ce_new / float(T)
        la_ref[...] = float(E) * jnp.sum(me * ce, axis=1, keepdims=True)


def _run_stage_a(xf, resf, w1, b1, w2, b2, ln_g, ln_b, wg):
    return pl.pallas_call(
        _stage_a,
        grid=(NBLK,),
        in_specs=[
            pl.BlockSpec((TB, D), lambda i: (i, 0)),
            pl.BlockSpec((B, TB // B, D), lambda i: (0, i, 0)),
            pl.BlockSpec((D, FH), lambda i: (0, 0)),
            pl.BlockSpec((1, FH), lambda i: (0, 0)),
            pl.BlockSpec((FH, D), lambda i: (0, 0)),
            pl.BlockSpec((1, D), lambda i: (0, 0)),
            pl.BlockSpec((1, D), lambda i: (0, 0)),
            pl.BlockSpec((1, D), lambda i: (0, 0)),
            pl.BlockSpec((D, E), lambda i: (0, 0)),
        ],
        out_specs=[
            pl.BlockSpec((TB, D), lambda i: (i, 0)),
            pl.BlockSpec((TB, D), lambda i: (i, 0)),
            pl.BlockSpec((TB, 1), lambda i: (i, 0)),
            pl.BlockSpec((TB, 1), lambda i: (i, 0)),
            pl.BlockSpec((TB, 1), lambda i: (i, 0)),
            pl.BlockSpec((1, 1), lambda i: (0, 0)),
        ],
        out_shape=[
            jax.ShapeDtypeStruct((T, D), jnp.float32),
            jax.ShapeDtypeStruct((T, D), jnp.float32),
            jax.ShapeDtypeStruct((T, 1), jnp.int32),
            jax.ShapeDtypeStruct((T, 1), jnp.int32),
            jax.ShapeDtypeStruct((T, 1), jnp.float32),
            jax.ShapeDtypeStruct((1, 1), jnp.float32),
        ],
        scratch_shapes=[
            pltpu.VMEM((1, E), jnp.float32),
            pltpu.VMEM((1, E), jnp.float32),
        ],
    )(xf, resf, w1, b1, w2, b2, ln_g, ln_b, wg)


def _sc_scatter(xn_hbm, idx_hbm, buf_hbm, idx_v, rows_v, sem):
    wid = lax.axis_index("s") * NC + lax.axis_index("c")
    for j in range(CPW):
        c = wid * CPW + j
        pltpu.sync_copy(idx_hbm.at[c], idx_v)
        pltpu.sync_copy(xn_hbm.at[pl.ds(c * CHUNK, CHUNK)], rows_v)
        pltpu.async_copy(rows_v, buf_hbm.at[idx_v], sem).wait()


@functools.lru_cache(maxsize=None)
def _sc_scatter_call():
    return pl.kernel(
        _sc_scatter,
        mesh=plsc.VectorSubcoreMesh(core_axis_name="c", subcore_axis_name="s"),
        out_type=jax.ShapeDtypeStruct((BUF_ROWS, D), jnp.float32),
        scratch_types=[
            pltpu.VMEM((CHUNK,), jnp.int32),
            pltpu.VMEM((CHUNK, D), jnp.float32),
            pltpu.SemaphoreType.DMA,
        ],
    )


def _sc_gather(oe_hbm, idx_hbm, out_hbm, idx_v, rows_v, sem):
    wid = lax.axis_index("s") * NC + lax.axis_index("c")
    for j in range(CPW):
        c = wid * CPW + j
        pltpu.sync_copy(idx_hbm.at[c], idx_v)
        pltpu.async_copy(oe_hbm.at[idx_v], rows_v, sem).wait()
        pltpu.sync_copy(rows_v, out_hbm.at[pl.ds(c * CHUNK, CHUNK)])


@functools.lru_cache(maxsize=None)
def _sc_gather_call():
    return pl.kernel(
        _sc_gather,
        mesh=plsc.VectorSubcoreMesh(core_axis_name="c", subcore_axis_name="s"),
        out_type=jax.ShapeDtypeStruct((T, D), jnp.float32),
        scratch_types=[
            pltpu.VMEM((CHUNK,), jnp.int32),
            pltpu.VMEM((CHUNK, D), jnp.float32),
            pltpu.SemaphoreType.DMA,
        ],
    )


def _stage_d(buf_ref, w1_ref, b1_ref, w2_ref, b2_ref, oe_ref):
    xb = buf_ref[...]
    w1e = w1_ref[...].reshape(D, DFF)
    w2e = w2_ref[...].reshape(DFF, D)
    h = jnp.maximum(
        jnp.dot(xb, w1e, preferred_element_type=jnp.float32)
        + b1_ref[...].reshape(1, DFF), 0.0)
    oe_ref[...] = (jnp.dot(h, w2e, preferred_element_type=jnp.float32)
                   + b2_ref[...].reshape(1, D))


def _run_stage_d(buf, we1, be1, we2, be2):
    return pl.pallas_call(
        _stage_d,
        grid=(E,),
        in_specs=[
            pl.BlockSpec((CAP, D), lambda e: (e, 0)),
            pl.BlockSpec((1, D, DFF), lambda e: (e, 0, 0)),
            pl.BlockSpec((1, 1, DFF), lambda e: (e, 0, 0)),
            pl.BlockSpec((1, DFF, D), lambda e: (e, 0, 0)),
            pl.BlockSpec((1, 1, D), lambda e: (e, 0, 0)),
        ],
        out_specs=[pl.BlockSpec((CAP, D), lambda e: (e, 0))],
        out_shape=[jax.ShapeDtypeStruct((E * CAP, D), jnp.float32)],
    )(buf, we1, be1, we2, be2)[0]


def _stage_f(gat_ref, cf_ref, sc_ref, out_ref):
    out_ref[...] = gat_ref[...] * cf_ref[...] + sc_ref[...]


def _run_stage_f(gathered, coef, shortcut):
    return pl.pallas_call(
        _stage_f,
        grid=(NBLK,),
        in_specs=[
            pl.BlockSpec((TB, D), lambda i: (i, 0)),
            pl.BlockSpec((TB, 1), lambda i: (i, 0)),
            pl.BlockSpec((TB, D), lambda i: (i, 0)),
        ],
        out_specs=[pl.BlockSpec((TB, D), lambda i: (i, 0))],
        out_shape=[jax.ShapeDtypeStruct((T, D), jnp.float32)],
    )(gathered, coef, shortcut)[0]


def kernel(x, residual, w1, b1, w2, b2, ln_g, ln_b, wg, we1, be1, we2, be2):
    xf = x.reshape(T, D)

    xn, shortcut, slot_s, slot_c, coef, la = _run_stage_a(
        xf, residual, w1, b1.reshape(1, FH), w2, b2.reshape(1, D),
        ln_g.reshape(1, D), ln_b.reshape(1, D), wg)

    buf = _sc_scatter_call()(xn, slot_s.reshape(NCHUNK, CHUNK))
    oe = _run_stage_d(buf, we1, be1.reshape(E, 1, DFF),
                      we2, be2.reshape(E, 1, D))
    gathered = _sc_gather_call()(oe, slot_c.reshape(NCHUNK, CHUNK))
    out = _run_stage_f(gathered, coef, shortcut)

    return out.reshape(S, B, D), la.reshape(())
